# baseline (speedup 1.0000x reference)
import functools

import jax
import jax.numpy as jnp
from jax import lax
from jax.experimental import pallas as pl
from jax.experimental.pallas import tpu as pltpu, tpu_sc as plsc


_NB = 512
_NW = 32
_QC = 32


def _topk_body(x1r, x2r, idxr, wr):
    nb = x1r.shape[2]
    s = x2r.shape[2]
    x1 = x1r[0]
    x2 = x2r[0]
    ones3 = jnp.ones((3, 1), jnp.float32)
    sq1 = lax.dot_general(x1 * x1, ones3, (((0,), (0,)), ((), ())),
                          precision=lax.Precision.HIGHEST)
    sq2 = jnp.sum(x2 * x2, axis=0, keepdims=True)
    cross = lax.dot_general(x1, x2, (((0,), (0,)), ((), ())))
    d = (-2.0 * cross + sq1) + sq2

    iota = lax.broadcasted_iota(jnp.int32, (nb, s), 1)
    work = d
    idxs = []
    vals = []
    for _ in range(3):
        vmin = jnp.min(work, axis=1, keepdims=True)
        hit = work == vmin
        ik = jnp.min(jnp.where(hit, iota, s), axis=1, keepdims=True)
        idxs.append(ik)
        vals.append(vmin)
        work = jnp.where(iota == ik, jnp.inf, work)

    r0 = 1.0 / (vals[0] + 1e-8)
    r1 = 1.0 / (vals[1] + 1e-8)
    r2 = 1.0 / (vals[2] + 1e-8)
    norm = r0 + r1 + r2

    gidx = jnp.concatenate(idxs, axis=1) + pl.program_id(0) * s
    idxr[...] = gidx[None]
    wr[...] = jnp.concatenate([r0 / norm, r1 / norm, r2 / norm], axis=1)[None]


def _sc_interp_body(idx_hbm, w_hbm, table_hbm, out_hbm,
                    idx_v, w_v, rows_v, out_v, sem):
    d = table_hbm.shape[1]
    qw = out_hbm.shape[0] // _NW
    nch = qw // _QC
    wid = lax.axis_index("s") * 2 + lax.axis_index("c")
    base_q = wid * qw

    def chunk(i, _):
        qb = base_q + i * _QC
        pltpu.sync_copy(idx_hbm.at[pl.ds(qb * 3, _QC * 3)], idx_v)
        pltpu.sync_copy(w_hbm.at[pl.ds(qb * 3, _QC * 3)],
                        w_v.at[pl.ds(0, _QC * 3)])
        pltpu.async_copy(table_hbm.at[idx_v], rows_v, sem).wait()

        def per_q(q, _):
            wvec = w_v[pl.ds(3 * q, 16)]
            w0 = wvec[0]
            w1 = wvec[1]
            w2 = wvec[2]
            for t in range(d // 16):
                sl = pl.ds(16 * t, 16)
                out_v[q, sl] = (rows_v[3 * q, sl] * w0
                                + rows_v[3 * q + 1, sl] * w1
                                + rows_v[3 * q + 2, sl] * w2)
            return 0

        lax.fori_loop(0, _QC, per_q, 0)
        pltpu.sync_copy(out_v, out_hbm.at[pl.ds(qb, _QC)])
        return 0

    lax.fori_loop(0, nch, chunk, 0)


def _y0_body(ir, p1r, w0r, b0r, y0r, s0r, ss0r):
    y0 = (lax.dot_general(w0r[...], ir[...], (((1,), (1,)), ((), ())))
          + lax.dot_general(w0r[...], p1r[0], (((1,), (0,)), ((), ())))
          + b0r[...])
    y0r[...] = y0[None]
    s0r[...] = jnp.sum(y0, axis=1, keepdims=True)[None]
    ss0r[...] = jnp.sum(y0 * y0, axis=1, keepdims=True)[None]


def _mid_body(y0r, a0r, c0r, w1r, b1r, y1r, s1r, ss1r):
    h = jnp.maximum(a0r[...] * y0r[0] + c0r[...], 0.0)
    y1 = lax.dot_general(w1r[...], h, (((1,), (0,)), ((), ()))) + b1r[...]
    y1r[...] = y1[None]
    s1r[...] = jnp.sum(y1, axis=1, keepdims=True)[None]
    ss1r[...] = jnp.sum(y1 * y1, axis=1, keepdims=True)[None]


def _out_body(y1r, a1r, c1r, outr):
    outr[...] = jnp.maximum(a1r[...] * y1r[0] + c1r[...], 0.0)[None]


def kernel(xyz1, xyz2, points1, points2, W0, b0, g0, be0, W1, b1, g1, be1):
    B, _, N = xyz1.shape
    S = xyz2.shape[2]
    D = points2.shape[1]
    C0 = W0.shape[0]
    C1 = W1.shape[0]
    NB = _NB
    NJ = N // NB
    BN = B * N
    M = float(BN)

    b0c = b0.reshape(C0, 1)
    b1c = b1.reshape(C1, 1)

    idxg, wts = pl.pallas_call(
        _topk_body,
        grid=(B, NJ),
        in_specs=[
            pl.BlockSpec((1, 3, NB), lambda b, j: (b, 0, j)),
            pl.BlockSpec((1, 3, S), lambda b, j: (b, 0, 0)),
        ],
        out_specs=[
            pl.BlockSpec((1, NB, 3), lambda b, j: (b * NJ + j, 0, 0)),
            pl.BlockSpec((1, NB, 3), lambda b, j: (b * NJ + j, 0, 0)),
        ],
        out_shape=[
            jax.ShapeDtypeStruct((B * NJ, NB, 3), jnp.int32),
            jax.ShapeDtypeStruct((B * NJ, NB, 3), jnp.float32),
        ],
    )(xyz1, xyz2)

    idx_flat = idxg.reshape(BN * 3)
    w_flat = wts.reshape(BN * 3)
    table = jnp.transpose(points2, (0, 2, 1)).reshape(B * S, D)

    mesh = plsc.VectorSubcoreMesh(core_axis_name="c", subcore_axis_name="s")
    interp = functools.partial(
        pl.kernel,
        mesh=mesh,
        out_type=jax.ShapeDtypeStruct((BN, D), jnp.float32),
        scratch_types=[
            pltpu.VMEM((_QC * 3,), jnp.int32),
            pltpu.VMEM((_QC * 3 + 16,), jnp.float32),
            pltpu.VMEM((_QC * 3, D), jnp.float32),
            pltpu.VMEM((_QC, D), jnp.float32),
            pltpu.SemaphoreType.DMA,
        ],
    )(_sc_interp_body)(idx_flat, w_flat, table)

    y0, s0, ss0 = pl.pallas_call(
        _y0_body,
        grid=(B, NJ),
        in_specs=[
            pl.BlockSpec((NB, D), lambda b, j: (b * NJ + j, 0)),
            pl.BlockSpec((1, D, NB), lambda b, j: (b, 0, j)),
            pl.BlockSpec((C0, D), lambda b, j: (0, 0)),
            pl.BlockSpec((C0, 1), lambda b, j: (0, 0)),
        ],
        out_specs=[
            pl.BlockSpec((1, C0, NB), lambda b, j: (b, 0, j)),
            pl.BlockSpec((1, C0, 1), lambda b, j: (b * NJ + j, 0, 0)),
            pl.BlockSpec((1, C0, 1), lambda b, j: (b * NJ + j, 0, 0)),
        ],
        out_shape=[
            jax.ShapeDtypeStruct((B, C0, N), jnp.float32),
            jax.ShapeDtypeStruct((B * NJ, C0, 1), jnp.float32),
            jax.ShapeDtypeStruct((B * NJ, C0, 1), jnp.float32),
        ],
    )(interp.reshape(BN, D), points1, W0, b0c)

    mean0 = jnp.sum(s0, axis=0) / M
    var0 = jnp.sum(ss0, axis=0) / M - mean0 * mean0
    a0 = g0.reshape(C0, 1) * lax.rsqrt(var0 + 1e-5)
    c0 = be0.reshape(C0, 1) - a0 * mean0

    y1, s1, ss1 = pl.pallas_call(
        _mid_body,
        grid=(B, NJ),
        in_specs=[
            pl.BlockSpec((1, C0, NB), lambda b, j: (b, 0, j)),
            pl.BlockSpec((C0, 1), lambda b, j: (0, 0)),
            pl.BlockSpec((C0, 1), lambda b, j: (0, 0)),
            pl.BlockSpec((C1, C0), lambda b, j: (0, 0)),
            pl.BlockSpec((C1, 1), lambda b, j: (0, 0)),
        ],
        out_specs=[
            pl.BlockSpec((1, C1, NB), lambda b, j: (b, 0, j)),
            pl.BlockSpec((1, C1, 1), lambda b, j: (b * NJ + j, 0, 0)),
            pl.BlockSpec((1, C1, 1), lambda b, j: (b * NJ + j, 0, 0)),
        ],
        out_shape=[
            jax.ShapeDtypeStruct((B, C1, N), jnp.float32),
            jax.ShapeDtypeStruct((B * NJ, C1, 1), jnp.float32),
            jax.ShapeDtypeStruct((B * NJ, C1, 1), jnp.float32),
        ],
    )(y0, a0, c0, W1, b1c)

    mean1 = jnp.sum(s1, axis=0) / M
    var1 = jnp.sum(ss1, axis=0) / M - mean1 * mean1
    a1 = g1.reshape(C1, 1) * lax.rsqrt(var1 + 1e-5)
    c1 = be1.reshape(C1, 1) - a1 * mean1

    out = pl.pallas_call(
        _out_body,
        grid=(B, NJ),
        in_specs=[
            pl.BlockSpec((1, C1, NB), lambda b, j: (b, 0, j)),
            pl.BlockSpec((C1, 1), lambda b, j: (0, 0)),
            pl.BlockSpec((C1, 1), lambda b, j: (0, 0)),
        ],
        out_specs=pl.BlockSpec((1, C1, NB), lambda b, j: (b, 0, j)),
        out_shape=jax.ShapeDtypeStruct((B, C1, N), jnp.float32),
    )(y1, a1, c1)

    return out


# speedup vs baseline: 1.1769x; 1.1769x over previous
import functools

import jax
import jax.numpy as jnp
from jax import lax
from jax.experimental import pallas as pl
from jax.experimental.pallas import tpu as pltpu, tpu_sc as plsc


_NB = 512
_NW = 32
_QC = 32


def _topk_body(x1r, x2r, idxr, wr):
    nb = x1r.shape[2]
    s = x2r.shape[2]
    x1 = x1r[0]
    x2 = x2r[0]
    ones3 = jnp.ones((3, 1), jnp.float32)
    sq1 = lax.dot_general(x1 * x1, ones3, (((0,), (0,)), ((), ())),
                          precision=lax.Precision.HIGHEST)
    sq2 = jnp.sum(x2 * x2, axis=0, keepdims=True)
    cross = lax.dot_general(x1, x2, (((0,), (0,)), ((), ())))
    d = (-2.0 * cross + sq1) + sq2

    iota = lax.broadcasted_iota(jnp.int32, (nb, s), 1)
    work = d
    idxs = []
    vals = []
    for _ in range(3):
        vmin = jnp.min(work, axis=1, keepdims=True)
        hit = work == vmin
        ik = jnp.min(jnp.where(hit, iota, s), axis=1, keepdims=True)
        idxs.append(ik)
        vals.append(vmin)
        work = jnp.where(iota == ik, jnp.inf, work)

    r0 = 1.0 / (vals[0] + 1e-8)
    r1 = 1.0 / (vals[1] + 1e-8)
    r2 = 1.0 / (vals[2] + 1e-8)
    norm = r0 + r1 + r2

    gidx = jnp.concatenate(idxs, axis=1) + pl.program_id(0) * s
    idxr[...] = gidx[None]
    wr[...] = jnp.concatenate([r0 / norm, r1 / norm, r2 / norm], axis=1)[None]


def _sc_interp_body(idx_hbm, w_hbm, table_hbm, out_hbm,
                    idx_v, w_v, rows0, rows1, out0, out1,
                    gsem0, gsem1, osem0, osem1):
    d = table_hbm.shape[1]
    qw = out_hbm.shape[0] // _NW
    nch = qw // _QC
    wid = lax.axis_index("s") * 2 + lax.axis_index("c")
    base_q = wid * qw
    pltpu.sync_copy(idx_hbm.at[pl.ds(base_q * 3, qw * 3)], idx_v)
    pltpu.sync_copy(w_hbm.at[pl.ds(base_q * 3, qw * 3)],
                    w_v.at[pl.ds(0, qw * 3)])
    pltpu.async_copy(table_hbm.at[idx_v.at[pl.ds(0, 3 * _QC)]], rows0, gsem0)

    def combine(rows_v, out_v, ci):
        def per_q(q, _):
            wvec = w_v[pl.ds(ci * 3 * _QC + 3 * q, 16)]
            w0 = wvec[0]
            w1 = wvec[1]
            w2 = wvec[2]
            for t in range(d // 16):
                sl = pl.ds(16 * t, 16)
                out_v[q, sl] = (rows_v[3 * q, sl] * w0
                                + rows_v[3 * q + 1, sl] * w1
                                + rows_v[3 * q + 2, sl] * w2)
            return 0

        lax.fori_loop(0, _QC, per_q, 0)

    def pair(p, _):
        for ph, (rows, gsem, nrows, ngsem, outb, osem) in enumerate([
                (rows0, gsem0, rows1, gsem1, out0, osem0),
                (rows1, gsem1, rows0, gsem0, out1, osem1)]):
            i = 2 * p + ph
            pltpu.make_async_copy(
                table_hbm.at[idx_v.at[pl.ds(0, 3 * _QC)]], rows, gsem).wait()

            @pl.when(i + 1 < nch)
            def _():
                pltpu.async_copy(
                    table_hbm.at[idx_v.at[pl.ds(3 * _QC * (i + 1), 3 * _QC)]],
                    nrows, ngsem)

            @pl.when(i >= 2)
            def _():
                pltpu.make_async_copy(
                    outb, out_hbm.at[pl.ds(0, _QC)], osem).wait()

            combine(rows, outb, i)
            pltpu.async_copy(outb, out_hbm.at[pl.ds(base_q + i * _QC, _QC)],
                             osem)
        return 0

    lax.fori_loop(0, nch // 2, pair, 0)
    pltpu.make_async_copy(out0, out_hbm.at[pl.ds(0, _QC)], osem0).wait()
    pltpu.make_async_copy(out1, out_hbm.at[pl.ds(0, _QC)], osem1).wait()


def _y0_body(ir, p1r, w0r, b0r, y0r, s0r, ss0r):
    y0 = (lax.dot_general(w0r[...], ir[...], (((1,), (1,)), ((), ())))
          + lax.dot_general(w0r[...], p1r[0], (((1,), (0,)), ((), ())))
          + b0r[...])
    y0r[...] = y0[None]
    s0r[...] = jnp.sum(y0, axis=1, keepdims=True)[None]
    ss0r[...] = jnp.sum(y0 * y0, axis=1, keepdims=True)[None]


def _mid_body(y0r, a0r, c0r, w1r, b1r, y1r, s1r, ss1r):
    h = jnp.maximum(a0r[...] * y0r[0] + c0r[...], 0.0)
    y1 = lax.dot_general(w1r[...], h, (((1,), (0,)), ((), ()))) + b1r[...]
    y1r[...] = y1[None]
    s1r[...] = jnp.sum(y1, axis=1, keepdims=True)[None]
    ss1r[...] = jnp.sum(y1 * y1, axis=1, keepdims=True)[None]


def _out_body(y1r, a1r, c1r, outr):
    outr[...] = jnp.maximum(a1r[...] * y1r[0] + c1r[...], 0.0)[None]


def kernel(xyz1, xyz2, points1, points2, W0, b0, g0, be0, W1, b1, g1, be1):
    B, _, N = xyz1.shape
    S = xyz2.shape[2]
    D = points2.shape[1]
    C0 = W0.shape[0]
    C1 = W1.shape[0]
    NB = _NB
    NJ = N // NB
    BN = B * N
    M = float(BN)

    b0c = b0.reshape(C0, 1)
    b1c = b1.reshape(C1, 1)

    idxg, wts = pl.pallas_call(
        _topk_body,
        grid=(B, NJ),
        in_specs=[
            pl.BlockSpec((1, 3, NB), lambda b, j: (b, 0, j)),
            pl.BlockSpec((1, 3, S), lambda b, j: (b, 0, 0)),
        ],
        out_specs=[
            pl.BlockSpec((1, NB, 3), lambda b, j: (b * NJ + j, 0, 0)),
            pl.BlockSpec((1, NB, 3), lambda b, j: (b * NJ + j, 0, 0)),
        ],
        out_shape=[
            jax.ShapeDtypeStruct((B * NJ, NB, 3), jnp.int32),
            jax.ShapeDtypeStruct((B * NJ, NB, 3), jnp.float32),
        ],
    )(xyz1, xyz2)

    idx_flat = idxg.reshape(BN * 3)
    w_flat = wts.reshape(BN * 3)
    table = jnp.transpose(points2, (0, 2, 1)).reshape(B * S, D)

    mesh = plsc.VectorSubcoreMesh(core_axis_name="c", subcore_axis_name="s")
    interp = functools.partial(
        pl.kernel,
        mesh=mesh,
        out_type=jax.ShapeDtypeStruct((BN, D), jnp.float32),
        scratch_types=[
            pltpu.VMEM((BN * 3 // _NW,), jnp.int32),
            pltpu.VMEM((BN * 3 // _NW + 16,), jnp.float32),
            pltpu.VMEM((_QC * 3, D), jnp.float32),
            pltpu.VMEM((_QC * 3, D), jnp.float32),
            pltpu.VMEM((_QC, D), jnp.float32),
            pltpu.VMEM((_QC, D), jnp.float32),
            pltpu.SemaphoreType.DMA,
            pltpu.SemaphoreType.DMA,
            pltpu.SemaphoreType.DMA,
            pltpu.SemaphoreType.DMA,
        ],
    )(_sc_interp_body)(idx_flat, w_flat, table)

    y0, s0, ss0 = pl.pallas_call(
        _y0_body,
        grid=(B, NJ),
        in_specs=[
            pl.BlockSpec((NB, D), lambda b, j: (b * NJ + j, 0)),
            pl.BlockSpec((1, D, NB), lambda b, j: (b, 0, j)),
            pl.BlockSpec((C0, D), lambda b, j: (0, 0)),
            pl.BlockSpec((C0, 1), lambda b, j: (0, 0)),
        ],
        out_specs=[
            pl.BlockSpec((1, C0, NB), lambda b, j: (b, 0, j)),
            pl.BlockSpec((1, C0, 1), lambda b, j: (b * NJ + j, 0, 0)),
            pl.BlockSpec((1, C0, 1), lambda b, j: (b * NJ + j, 0, 0)),
        ],
        out_shape=[
            jax.ShapeDtypeStruct((B, C0, N), jnp.float32),
            jax.ShapeDtypeStruct((B * NJ, C0, 1), jnp.float32),
            jax.ShapeDtypeStruct((B * NJ, C0, 1), jnp.float32),
        ],
    )(interp.reshape(BN, D), points1, W0, b0c)

    mean0 = jnp.sum(s0, axis=0) / M
    var0 = jnp.sum(ss0, axis=0) / M - mean0 * mean0
    a0 = g0.reshape(C0, 1) * lax.rsqrt(var0 + 1e-5)
    c0 = be0.reshape(C0, 1) - a0 * mean0

    y1, s1, ss1 = pl.pallas_call(
        _mid_body,
        grid=(B, NJ),
        in_specs=[
            pl.BlockSpec((1, C0, NB), lambda b, j: (b, 0, j)),
            pl.BlockSpec((C0, 1), lambda b, j: (0, 0)),
            pl.BlockSpec((C0, 1), lambda b, j: (0, 0)),
            pl.BlockSpec((C1, C0), lambda b, j: (0, 0)),
            pl.BlockSpec((C1, 1), lambda b, j: (0, 0)),
        ],
        out_specs=[
            pl.BlockSpec((1, C1, NB), lambda b, j: (b, 0, j)),
            pl.BlockSpec((1, C1, 1), lambda b, j: (b * NJ + j, 0, 0)),
            pl.BlockSpec((1, C1, 1), lambda b, j: (b * NJ + j, 0, 0)),
        ],
        out_shape=[
            jax.ShapeDtypeStruct((B, C1, N), jnp.float32),
            jax.ShapeDtypeStruct((B * NJ, C1, 1), jnp.float32),
            jax.ShapeDtypeStruct((B * NJ, C1, 1), jnp.float32),
        ],
    )(y0, a0, c0, W1, b1c)

    mean1 = jnp.sum(s1, axis=0) / M
    var1 = jnp.sum(ss1, axis=0) / M - mean1 * mean1
    a1 = g1.reshape(C1, 1) * lax.rsqrt(var1 + 1e-5)
    c1 = be1.reshape(C1, 1) - a1 * mean1

    out = pl.pallas_call(
        _out_body,
        grid=(B, NJ),
        in_specs=[
            pl.BlockSpec((1, C1, NB), lambda b, j: (b, 0, j)),
            pl.BlockSpec((C1, 1), lambda b, j: (0, 0)),
            pl.BlockSpec((C1, 1), lambda b, j: (0, 0)),
        ],
        out_specs=pl.BlockSpec((1, C1, NB), lambda b, j: (b, 0, j)),
        out_shape=jax.ShapeDtypeStruct((B, C1, N), jnp.float32),
    )(y1, a1, c1)

    return out


# speedup vs baseline: 1.5386x; 1.3073x over previous
import functools

import jax
import jax.numpy as jnp
from jax import lax
from jax.experimental import pallas as pl
from jax.experimental.pallas import tpu as pltpu, tpu_sc as plsc


_NB = 512
_NW = 32
_QC = 32


def _topk_body(x1r, x2r, idxr, wr):
    nb = x1r.shape[2]
    s = x2r.shape[2]
    x1 = x1r[0]
    x2 = x2r[0]
    ones3 = jnp.ones((3, 1), jnp.float32)
    sq1 = lax.dot_general(x1 * x1, ones3, (((0,), (0,)), ((), ())),
                          precision=lax.Precision.HIGHEST)
    sq2 = jnp.sum(x2 * x2, axis=0, keepdims=True)
    cross = lax.dot_general(x1, x2, (((0,), (0,)), ((), ())))
    d = (-2.0 * cross + sq1) + sq2

    iota = lax.broadcasted_iota(jnp.int32, (nb, s), 1)
    work = d
    idxs = []
    vals = []
    for _ in range(3):
        vmin = jnp.min(work, axis=1, keepdims=True)
        hit = work == vmin
        ik = jnp.min(jnp.where(hit, iota, s), axis=1, keepdims=True)
        idxs.append(ik)
        vals.append(vmin)
        work = jnp.where(iota == ik, jnp.inf, work)

    r0 = 1.0 / (vals[0] + 1e-8)
    r1 = 1.0 / (vals[1] + 1e-8)
    r2 = 1.0 / (vals[2] + 1e-8)
    norm = r0 + r1 + r2

    gidx = jnp.concatenate(idxs, axis=1) + pl.program_id(0) * s
    idxr[...] = gidx[None]
    wr[...] = jnp.concatenate([r0 / norm, r1 / norm, r2 / norm], axis=1)[None]


def _sc_interp_body(idx_hbm, w_hbm, table_hbm, out_hbm,
                    idx_v, w_v, rows0, rows1, out0, out1,
                    gsem0, gsem1, osem0, osem1):
    d = table_hbm.shape[1]
    qw = out_hbm.shape[0] // _NW
    nch = qw // _QC
    wid = lax.axis_index("s") * 2 + lax.axis_index("c")
    base_q = wid * qw
    pltpu.sync_copy(idx_hbm.at[pl.ds(base_q * 3, qw * 3)], idx_v)
    pltpu.sync_copy(w_hbm.at[pl.ds(base_q * 3, qw * 3)],
                    w_v.at[pl.ds(0, qw * 3)])
    pltpu.async_copy(table_hbm.at[idx_v.at[pl.ds(0, 3 * _QC)]], rows0, gsem0)

    def combine(rows_v, out_v, ci):
        def per_q(qq, _):
            for u in range(2):
                q = 2 * qq + u
                wvec = w_v[pl.ds(ci * 3 * _QC + 3 * q, 16)]
                w0 = wvec[0]
                w1 = wvec[1]
                w2 = wvec[2]
                for t in range(d // 16):
                    sl = pl.ds(16 * t, 16)
                    out_v[q, sl] = (rows_v[3 * q, sl] * w0
                                    + rows_v[3 * q + 1, sl] * w1
                                    + rows_v[3 * q + 2, sl] * w2)
            return 0

        lax.fori_loop(0, _QC // 2, per_q, 0)

    def pair(p, _):
        for ph, (rows, gsem, nrows, ngsem, outb, osem) in enumerate([
                (rows0, gsem0, rows1, gsem1, out0, osem0),
                (rows1, gsem1, rows0, gsem0, out1, osem1)]):
            i = 2 * p + ph
            pltpu.make_async_copy(
                table_hbm.at[idx_v.at[pl.ds(0, 3 * _QC)]], rows, gsem).wait()

            @pl.when(i + 1 < nch)
            def _():
                pltpu.async_copy(
                    table_hbm.at[idx_v.at[pl.ds(3 * _QC * (i + 1), 3 * _QC)]],
                    nrows, ngsem)

            @pl.when(i >= 2)
            def _():
                pltpu.make_async_copy(
                    outb, out_hbm.at[pl.ds(0, _QC)], osem).wait()

            combine(rows, outb, i)
            pltpu.async_copy(outb, out_hbm.at[pl.ds(base_q + i * _QC, _QC)],
                             osem)
        return 0

    lax.fori_loop(0, nch // 2, pair, 0)
    pltpu.make_async_copy(out0, out_hbm.at[pl.ds(0, _QC)], osem0).wait()
    pltpu.make_async_copy(out1, out_hbm.at[pl.ds(0, _QC)], osem1).wait()


def _y0_body(ir, p1r, w0r, b0r, y0r, s0r, ss0r):
    y0 = (lax.dot_general(w0r[...], ir[...], (((1,), (1,)), ((), ())))
          + lax.dot_general(w0r[...], p1r[0], (((1,), (0,)), ((), ())))
          + b0r[...])
    y0r[...] = y0[None]
    s0r[...] = jnp.sum(y0, axis=1, keepdims=True)[None]
    ss0r[...] = jnp.sum(y0 * y0, axis=1, keepdims=True)[None]


def _mid_body(y0r, a0r, c0r, w1r, b1r, y1r, s1r, ss1r):
    h = jnp.maximum(a0r[...] * y0r[0] + c0r[...], 0.0)
    y1 = lax.dot_general(w1r[...], h, (((1,), (0,)), ((), ()))) + b1r[...]
    y1r[...] = y1[None]
    s1r[...] = jnp.sum(y1, axis=1, keepdims=True)[None]
    ss1r[...] = jnp.sum(y1 * y1, axis=1, keepdims=True)[None]


def _out_body(y1r, a1r, c1r, outr):
    outr[...] = jnp.maximum(a1r[...] * y1r[0] + c1r[...], 0.0)[None]


def kernel(xyz1, xyz2, points1, points2, W0, b0, g0, be0, W1, b1, g1, be1):
    B, _, N = xyz1.shape
    S = xyz2.shape[2]
    D = points2.shape[1]
    C0 = W0.shape[0]
    C1 = W1.shape[0]
    NB = _NB
    NJ = N // NB
    BN = B * N
    M = float(BN)

    b0c = b0.reshape(C0, 1)
    b1c = b1.reshape(C1, 1)

    table = jnp.transpose(points2, (0, 2, 1)).reshape(B * S, D)
    mesh = plsc.VectorSubcoreMesh(core_axis_name="c", subcore_axis_name="s")

    def _half(xyz1_h, xyz2_h, table_h):
        bh = xyz1_h.shape[0]
        bnh = bh * N
        idxg, wts = pl.pallas_call(
            _topk_body,
            grid=(bh, NJ),
            in_specs=[
                pl.BlockSpec((1, 3, NB), lambda b, j: (b, 0, j)),
                pl.BlockSpec((1, 3, S), lambda b, j: (b, 0, 0)),
            ],
            out_specs=[
                pl.BlockSpec((1, NB, 3), lambda b, j: (b * NJ + j, 0, 0)),
                pl.BlockSpec((1, NB, 3), lambda b, j: (b * NJ + j, 0, 0)),
            ],
            out_shape=[
                jax.ShapeDtypeStruct((bh * NJ, NB, 3), jnp.int32),
                jax.ShapeDtypeStruct((bh * NJ, NB, 3), jnp.float32),
            ],
        )(xyz1_h, xyz2_h)

        return functools.partial(
            pl.kernel,
            mesh=mesh,
            out_type=jax.ShapeDtypeStruct((bnh, D), jnp.float32),
            scratch_types=[
                pltpu.VMEM((bnh * 3 // _NW,), jnp.int32),
                pltpu.VMEM((bnh * 3 // _NW + 16,), jnp.float32),
                pltpu.VMEM((_QC * 3, D), jnp.float32),
                pltpu.VMEM((_QC * 3, D), jnp.float32),
                pltpu.VMEM((_QC, D), jnp.float32),
                pltpu.VMEM((_QC, D), jnp.float32),
                pltpu.SemaphoreType.DMA,
                pltpu.SemaphoreType.DMA,
                pltpu.SemaphoreType.DMA,
                pltpu.SemaphoreType.DMA,
            ],
        )(_sc_interp_body)(idxg.reshape(bnh * 3), wts.reshape(bnh * 3),
                           table_h)

    NSPLIT = 4
    BH = B // NSPLIT
    interps = [
        _half(xyz1[h * BH:(h + 1) * BH], xyz2[h * BH:(h + 1) * BH],
              table[h * BH * S:(h + 1) * BH * S])
        for h in range(NSPLIT)
    ]

    NB2 = 2048
    NJ2 = N // NB2

    def _y0_half(interp_h, points1_h):
        bh = points1_h.shape[0]
        return pl.pallas_call(
            _y0_body,
            grid=(bh, NJ2),
            in_specs=[
                pl.BlockSpec((NB2, D), lambda b, j: (b * NJ2 + j, 0)),
                pl.BlockSpec((1, D, NB2), lambda b, j: (b, 0, j)),
                pl.BlockSpec((C0, D), lambda b, j: (0, 0)),
                pl.BlockSpec((C0, 1), lambda b, j: (0, 0)),
            ],
            out_specs=[
                pl.BlockSpec((1, C0, NB2), lambda b, j: (b, 0, j)),
                pl.BlockSpec((1, C0, 1), lambda b, j: (b * NJ2 + j, 0, 0)),
                pl.BlockSpec((1, C0, 1), lambda b, j: (b * NJ2 + j, 0, 0)),
            ],
            out_shape=[
                jax.ShapeDtypeStruct((bh, C0, N), jnp.float32),
                jax.ShapeDtypeStruct((bh * NJ2, C0, 1), jnp.float32),
                jax.ShapeDtypeStruct((bh * NJ2, C0, 1), jnp.float32),
            ],
        )(interp_h, points1_h, W0, b0c)

    y0s = [_y0_half(interps[h], points1[h * BH:(h + 1) * BH])
           for h in range(NSPLIT)]

    mean0 = sum(jnp.sum(r[1], axis=0) for r in y0s) / M
    var0 = sum(jnp.sum(r[2], axis=0) for r in y0s) / M - mean0 * mean0
    a0 = g0.reshape(C0, 1) * lax.rsqrt(var0 + 1e-5)
    c0 = be0.reshape(C0, 1) - a0 * mean0

    def _mid_half(y0_h):
        bh = y0_h.shape[0]
        return pl.pallas_call(
            _mid_body,
            grid=(bh, NJ2),
            in_specs=[
                pl.BlockSpec((1, C0, NB2), lambda b, j: (b, 0, j)),
                pl.BlockSpec((C0, 1), lambda b, j: (0, 0)),
                pl.BlockSpec((C0, 1), lambda b, j: (0, 0)),
                pl.BlockSpec((C1, C0), lambda b, j: (0, 0)),
                pl.BlockSpec((C1, 1), lambda b, j: (0, 0)),
            ],
            out_specs=[
                pl.BlockSpec((1, C1, NB2), lambda b, j: (b, 0, j)),
                pl.BlockSpec((1, C1, 1), lambda b, j: (b * NJ2 + j, 0, 0)),
                pl.BlockSpec((1, C1, 1), lambda b, j: (b * NJ2 + j, 0, 0)),
            ],
            out_shape=[
                jax.ShapeDtypeStruct((bh, C1, N), jnp.float32),
                jax.ShapeDtypeStruct((bh * NJ2, C1, 1), jnp.float32),
                jax.ShapeDtypeStruct((bh * NJ2, C1, 1), jnp.float32),
            ],
        )(y0_h, a0, c0, W1, b1c)

    y1s = [_mid_half(r[0]) for r in y0s]

    mean1 = sum(jnp.sum(r[1], axis=0) for r in y1s) / M
    var1 = sum(jnp.sum(r[2], axis=0) for r in y1s) / M - mean1 * mean1
    a1 = g1.reshape(C1, 1) * lax.rsqrt(var1 + 1e-5)
    c1 = be1.reshape(C1, 1) - a1 * mean1

    def _out_half(y1_h):
        bh = y1_h.shape[0]
        return pl.pallas_call(
            _out_body,
            grid=(bh, NJ2),
            in_specs=[
                pl.BlockSpec((1, C1, NB2), lambda b, j: (b, 0, j)),
                pl.BlockSpec((C1, 1), lambda b, j: (0, 0)),
                pl.BlockSpec((C1, 1), lambda b, j: (0, 0)),
            ],
            out_specs=pl.BlockSpec((1, C1, NB2), lambda b, j: (b, 0, j)),
            out_shape=jax.ShapeDtypeStruct((bh, C1, N), jnp.float32),
        )(y1_h, a1, c1)

    return jnp.concatenate([_out_half(r[0]) for r in y1s], axis=0)


# speedup vs baseline: 1.5636x; 1.0163x over previous
import functools

import jax
import jax.numpy as jnp
from jax import lax
from jax.experimental import pallas as pl
from jax.experimental.pallas import tpu as pltpu, tpu_sc as plsc


_NB = 512
_NW = 32
_QC = 32


def _topk_body(x1r, x2r, idxr, wr):
    nb = x1r.shape[2]
    s = x2r.shape[2]
    x1 = x1r[0]
    x2 = x2r[0]
    ones3 = jnp.ones((3, 1), jnp.float32)
    sq1 = lax.dot_general(x1 * x1, ones3, (((0,), (0,)), ((), ())),
                          precision=lax.Precision.HIGHEST)
    sq2 = jnp.sum(x2 * x2, axis=0, keepdims=True)
    cross = lax.dot_general(x1, x2, (((0,), (0,)), ((), ())))
    d = (-2.0 * cross + sq1) + sq2

    iota = lax.broadcasted_iota(jnp.int32, (nb, s), 1)
    work = d
    idxs = []
    vals = []
    for _ in range(3):
        vmin = jnp.min(work, axis=1, keepdims=True)
        hit = work == vmin
        ik = jnp.min(jnp.where(hit, iota, s), axis=1, keepdims=True)
        idxs.append(ik)
        vals.append(vmin)
        work = jnp.where(iota == ik, jnp.inf, work)

    r0 = 1.0 / (vals[0] + 1e-8)
    r1 = 1.0 / (vals[1] + 1e-8)
    r2 = 1.0 / (vals[2] + 1e-8)
    norm = r0 + r1 + r2

    gidx = jnp.concatenate(idxs, axis=1) + pl.program_id(0) * s
    idxr[...] = gidx[None]
    wr[...] = jnp.concatenate([r0 / norm, r1 / norm, r2 / norm], axis=1)[None]


def _sc_interp_body(idx_hbm, w_hbm, table_hbm, out_hbm,
                    idx_v, w_v, rows0, rows1, out0, out1,
                    gsem0, gsem1, osem0, osem1):
    d = table_hbm.shape[1]
    qw = out_hbm.shape[0] // _NW
    nch = qw // _QC
    wid = lax.axis_index("s") * 2 + lax.axis_index("c")
    base_q = wid * qw
    pltpu.sync_copy(idx_hbm.at[pl.ds(base_q * 3, qw * 3)], idx_v)
    pltpu.sync_copy(w_hbm.at[pl.ds(base_q * 3, qw * 3)],
                    w_v.at[pl.ds(0, qw * 3)])
    pltpu.async_copy(table_hbm.at[idx_v.at[pl.ds(0, 3 * _QC)]], rows0, gsem0)

    def combine(rows_v, out_v, ci):
        def per_q(qq, _):
            for u in range(2):
                q = 2 * qq + u
                wvec = w_v[pl.ds(ci * 3 * _QC + 3 * q, 16)]
                w0 = wvec[0]
                w1 = wvec[1]
                w2 = wvec[2]
                for t in range(d // 16):
                    sl = pl.ds(16 * t, 16)
                    out_v[q, sl] = (rows_v[3 * q, sl] * w0
                                    + rows_v[3 * q + 1, sl] * w1
                                    + rows_v[3 * q + 2, sl] * w2)
            return 0

        lax.fori_loop(0, _QC // 2, per_q, 0)

    def pair(p, _):
        for ph, (rows, gsem, nrows, ngsem, outb, osem) in enumerate([
                (rows0, gsem0, rows1, gsem1, out0, osem0),
                (rows1, gsem1, rows0, gsem0, out1, osem1)]):
            i = 2 * p + ph
            pltpu.make_async_copy(
                table_hbm.at[idx_v.at[pl.ds(0, 3 * _QC)]], rows, gsem).wait()

            @pl.when(i + 1 < nch)
            def _():
                pltpu.async_copy(
                    table_hbm.at[idx_v.at[pl.ds(3 * _QC * (i + 1), 3 * _QC)]],
                    nrows, ngsem)

            @pl.when(i >= 2)
            def _():
                pltpu.make_async_copy(
                    outb, out_hbm.at[pl.ds(0, _QC)], osem).wait()

            combine(rows, outb, i)
            pltpu.async_copy(outb, out_hbm.at[pl.ds(base_q + i * _QC, _QC)],
                             osem)
        return 0

    lax.fori_loop(0, nch // 2, pair, 0)
    pltpu.make_async_copy(out0, out_hbm.at[pl.ds(0, _QC)], osem0).wait()
    pltpu.make_async_copy(out1, out_hbm.at[pl.ds(0, _QC)], osem1).wait()


def _y0_body(ir, p1r, w0r, b0r, y0r, s0r, ss0r):
    y0 = (lax.dot_general(w0r[...], ir[...], (((1,), (1,)), ((), ())))
          + lax.dot_general(w0r[...], p1r[0], (((1,), (0,)), ((), ())))
          + b0r[...])
    y0r[...] = y0[None]
    s0r[...] = jnp.sum(y0, axis=1, keepdims=True)[None]
    ss0r[...] = jnp.sum(y0 * y0, axis=1, keepdims=True)[None]


def _mid_body(y0r, a0r, c0r, w1r, b1r, y1r, s1r, ss1r):
    h = jnp.maximum(a0r[...] * y0r[0] + c0r[...], 0.0)
    y1 = lax.dot_general(w1r[...], h, (((1,), (0,)), ((), ()))) + b1r[...]
    y1r[...] = y1[None]
    s1r[...] = jnp.sum(y1, axis=1, keepdims=True)[None]
    ss1r[...] = jnp.sum(y1 * y1, axis=1, keepdims=True)[None]


def _out_body(y1r, a1r, c1r, outr):
    outr[...] = jnp.maximum(a1r[...] * y1r[0] + c1r[...], 0.0)[None]


def kernel(xyz1, xyz2, points1, points2, W0, b0, g0, be0, W1, b1, g1, be1):
    B, _, N = xyz1.shape
    S = xyz2.shape[2]
    D = points2.shape[1]
    C0 = W0.shape[0]
    C1 = W1.shape[0]
    NB = _NB
    NJ = N // NB
    BN = B * N
    M = float(BN)

    b0c = b0.reshape(C0, 1)
    b1c = b1.reshape(C1, 1)

    table = jnp.transpose(points2, (0, 2, 1)).reshape(B * S, D)
    mesh = plsc.VectorSubcoreMesh(core_axis_name="c", subcore_axis_name="s")

    def _half(xyz1_h, xyz2_h, table_h):
        bh = xyz1_h.shape[0]
        bnh = bh * N
        idxg, wts = pl.pallas_call(
            _topk_body,
            grid=(bh, NJ),
            in_specs=[
                pl.BlockSpec((1, 3, NB), lambda b, j: (b, 0, j)),
                pl.BlockSpec((1, 3, S), lambda b, j: (b, 0, 0)),
            ],
            out_specs=[
                pl.BlockSpec((1, NB, 3), lambda b, j: (b * NJ + j, 0, 0)),
                pl.BlockSpec((1, NB, 3), lambda b, j: (b * NJ + j, 0, 0)),
            ],
            out_shape=[
                jax.ShapeDtypeStruct((bh * NJ, NB, 3), jnp.int32),
                jax.ShapeDtypeStruct((bh * NJ, NB, 3), jnp.float32),
            ],
        )(xyz1_h, xyz2_h)

        return functools.partial(
            pl.kernel,
            mesh=mesh,
            out_type=jax.ShapeDtypeStruct((bnh, D), jnp.float32),
            scratch_types=[
                pltpu.VMEM((bnh * 3 // _NW,), jnp.int32),
                pltpu.VMEM((bnh * 3 // _NW + 16,), jnp.float32),
                pltpu.VMEM((_QC * 3, D), jnp.float32),
                pltpu.VMEM((_QC * 3, D), jnp.float32),
                pltpu.VMEM((_QC, D), jnp.float32),
                pltpu.VMEM((_QC, D), jnp.float32),
                pltpu.SemaphoreType.DMA,
                pltpu.SemaphoreType.DMA,
                pltpu.SemaphoreType.DMA,
                pltpu.SemaphoreType.DMA,
            ],
        )(_sc_interp_body)(idxg.reshape(bnh * 3), wts.reshape(bnh * 3),
                           table_h)

    NSPLIT = 2
    BH = B // NSPLIT
    interps = [
        _half(xyz1[h * BH:(h + 1) * BH], xyz2[h * BH:(h + 1) * BH],
              table[h * BH * S:(h + 1) * BH * S])
        for h in range(NSPLIT)
    ]

    NB2 = 2048
    NJ2 = N // NB2

    def _y0_half(interp_h, points1_h):
        bh = points1_h.shape[0]
        return pl.pallas_call(
            _y0_body,
            grid=(bh, NJ2),
            in_specs=[
                pl.BlockSpec((NB2, D), lambda b, j: (b * NJ2 + j, 0)),
                pl.BlockSpec((1, D, NB2), lambda b, j: (b, 0, j)),
                pl.BlockSpec((C0, D), lambda b, j: (0, 0)),
                pl.BlockSpec((C0, 1), lambda b, j: (0, 0)),
            ],
            out_specs=[
                pl.BlockSpec((1, C0, NB2), lambda b, j: (b, 0, j)),
                pl.BlockSpec((1, C0, 1), lambda b, j: (b * NJ2 + j, 0, 0)),
                pl.BlockSpec((1, C0, 1), lambda b, j: (b * NJ2 + j, 0, 0)),
            ],
            out_shape=[
                jax.ShapeDtypeStruct((bh, C0, N), jnp.float32),
                jax.ShapeDtypeStruct((bh * NJ2, C0, 1), jnp.float32),
                jax.ShapeDtypeStruct((bh * NJ2, C0, 1), jnp.float32),
            ],
        )(interp_h, points1_h, W0, b0c)

    y0s = [_y0_half(interps[h], points1[h * BH:(h + 1) * BH])
           for h in range(NSPLIT)]

    mean0 = sum(jnp.sum(r[1], axis=0) for r in y0s) / M
    var0 = sum(jnp.sum(r[2], axis=0) for r in y0s) / M - mean0 * mean0
    a0 = g0.reshape(C0, 1) * lax.rsqrt(var0 + 1e-5)
    c0 = be0.reshape(C0, 1) - a0 * mean0

    def _mid_half(y0_h):
        bh = y0_h.shape[0]
        return pl.pallas_call(
            _mid_body,
            grid=(bh, NJ2),
            in_specs=[
                pl.BlockSpec((1, C0, NB2), lambda b, j: (b, 0, j)),
                pl.BlockSpec((C0, 1), lambda b, j: (0, 0)),
                pl.BlockSpec((C0, 1), lambda b, j: (0, 0)),
                pl.BlockSpec((C1, C0), lambda b, j: (0, 0)),
                pl.BlockSpec((C1, 1), lambda b, j: (0, 0)),
            ],
            out_specs=[
                pl.BlockSpec((1, C1, NB2), lambda b, j: (b, 0, j)),
                pl.BlockSpec((1, C1, 1), lambda b, j: (b * NJ2 + j, 0, 0)),
                pl.BlockSpec((1, C1, 1), lambda b, j: (b * NJ2 + j, 0, 0)),
            ],
            out_shape=[
                jax.ShapeDtypeStruct((bh, C1, N), jnp.float32),
                jax.ShapeDtypeStruct((bh * NJ2, C1, 1), jnp.float32),
                jax.ShapeDtypeStruct((bh * NJ2, C1, 1), jnp.float32),
            ],
        )(y0_h, a0, c0, W1, b1c)

    y1s = [_mid_half(r[0]) for r in y0s]

    mean1 = sum(jnp.sum(r[1], axis=0) for r in y1s) / M
    var1 = sum(jnp.sum(r[2], axis=0) for r in y1s) / M - mean1 * mean1
    a1 = g1.reshape(C1, 1) * lax.rsqrt(var1 + 1e-5)
    c1 = be1.reshape(C1, 1) - a1 * mean1

    def _out_half(y1_h):
        bh = y1_h.shape[0]
        return pl.pallas_call(
            _out_body,
            grid=(bh, NJ2),
            in_specs=[
                pl.BlockSpec((1, C1, NB2), lambda b, j: (b, 0, j)),
                pl.BlockSpec((C1, 1), lambda b, j: (0, 0)),
                pl.BlockSpec((C1, 1), lambda b, j: (0, 0)),
            ],
            out_specs=pl.BlockSpec((1, C1, NB2), lambda b, j: (b, 0, j)),
            out_shape=jax.ShapeDtypeStruct((bh, C1, N), jnp.float32),
        )(y1_h, a1, c1)

    return jnp.concatenate([_out_half(r[0]) for r in y1s], axis=0)
